# Initial kernel scaffold; baseline (speedup 1.0000x reference)
import jax
import jax.numpy as jnp
from jax.experimental import pallas as pl


def kernel(X, tables, W_cat, W_gen, W_fc, beta):
    raise NotImplementedError("write your pallas kernel here")


# speedup vs baseline: 1.7680x; 1.7680x over previous
import functools

import numpy as np
import jax
import jax.numpy as jnp
from jax.experimental import pallas as pl
from jax.experimental.pallas import tpu as pltpu
from jax.experimental.pallas import tpu_sc as plsc

_B, _F, _V, _D = 4096, 26, 100000, 16
_I, _J = np.triu_indices(_F, k=1)
_P = _I.shape[0]

_T = [np.where(np.arange(_P) % 4 == t)[0] for t in range(4)]
_PERM = np.concatenate(_T)
_N0, _N1, _N2, _N3 = (len(t) for t in _T)

_S0 = np.zeros((_F, _N0), np.float32)
for _q, _p in enumerate(_T[0]):
    _S0[_I[_p], _q] += 1.0
    _S0[_J[_p], _q] += 1.0


def _onehot(plist, sel):
    m = np.zeros((_F, len(plist)), np.float32)
    for q, p in enumerate(plist):
        m[sel[p], q] = 1.0
    return m


_SI1, _SJ1 = _onehot(_T[1], _I), _onehot(_T[1], _J)
_SI2, _SJ2 = _onehot(_T[2], _I), _onehot(_T[2], _J)
_SI3, _SJ3 = _onehot(_T[3], _I), _onehot(_T[3], _J)
_S2SEL = np.zeros((_N2 * _D, _N2), np.float32)
for _q in range(_N2):
    _S2SEL[_q * _D:(_q + 1) * _D, _q] = 1.0

_BS = 512
_NW = 32
_GW = 128
_BPW = (_B * _F) // _NW
_NCH = _BPW // _GW
_GD = 128
_RPG = _GD // _D


def _sc_gather(table128, idx3):
    mesh = plsc.VectorSubcoreMesh(core_axis_name="c", subcore_axis_name="s")

    @functools.partial(
        pl.kernel, mesh=mesh,
        out_type=jax.ShapeDtypeStruct((_B * _F, _GD), jnp.float32),
        scratch_types=[
            pltpu.VMEM((_NCH, _GW), jnp.int32),
            pltpu.VMEM((_GW, _GD), jnp.float32),
            pltpu.SemaphoreType.DMA,
        ],
    )
    def kern(table_hbm, idx_hbm, out_hbm, idx_v, rows_v, sem):
        wid = jax.lax.axis_index("s") * 2 + jax.lax.axis_index("c")
        pltpu.sync_copy(idx_hbm.at[wid], idx_v)

        @pl.loop(0, _NCH)
        def _(j):
            pltpu.async_copy(table_hbm.at[idx_v.at[j]], rows_v, sem).wait()
            pltpu.sync_copy(
                rows_v, out_hbm.at[pl.ds(wid * _BPW + j * _GW, _GW)])

    return kern(table128, idx3)


def _interact_kernel(g0, g1, g2, g3, g4, g5, g6, g7, rem_ref,
                     w0_ref, d1l_ref, d1r_ref, w2_ref, s2w_ref,
                     d3l_ref, d3r_ref, sum1_ref, out_ref):
    gk = (g0, g1, g2, g3, g4, g5, g6, g7)
    rem = rem_ref[...]
    e2 = jnp.zeros((_BS, _F * _D), jnp.float32)
    for k in range(_RPG):
        e2 = e2 + gk[k][...] * (rem == k).astype(jnp.float32)

    s0 = jnp.dot(e2, w0_ref[...])
    a = jnp.dot(e2, d1l_ref[...])
    b = jnp.dot(e2, d1r_ref[...])
    s1 = jnp.dot(a * b, sum1_ref[...])
    c = jnp.maximum(jnp.dot(e2, w2_ref[...]), 0.0)
    s2 = jnp.dot(c, s2w_ref[...])
    p = jnp.dot(e2, d3l_ref[...])
    q = jnp.dot(e2, d3r_ref[...])
    s3 = jnp.dot(p * q, sum1_ref[...])

    out_ref[...] = jnp.concatenate([s0, s1, s2, s3], axis=1)


def _final_kernel(s_ref, beta_ref, out_ref):
    s = s_ref[...]
    mean = jnp.mean(s, axis=0, keepdims=True)
    var = jnp.mean((s - mean) ** 2, axis=0, keepdims=True)
    g = jnp.tanh(beta_ref[...]) / jnp.sqrt(var + 1e-3)
    out_ref[...] = (jnp.sum(s * g, axis=1, keepdims=True)
                    - jnp.sum(g * mean))


def kernel(X, tables, W_cat, W_gen, W_fc, beta):
    table128 = tables.reshape(_F * _V * _D // _GD, _GD)
    phys_idx = (X // _RPG
                + (jnp.arange(_F, dtype=jnp.int32) * (_V // _RPG))[None, :]
                ).reshape(_NW, _NCH, _GW)
    rem = X % _RPG
    g = _sc_gather(table128, phys_idx)
    g3 = g.reshape(_B, _F, _GD)
    gks = [g3[:, :, k * _D:(k + 1) * _D].reshape(_B, _F * _D)
           for k in range(_RPG)]
    rem416 = jnp.repeat(rem, _D, axis=1)

    w = W_fc[0]
    v = w @ W_gen
    A = W_cat[:, :_D]
    Bm = W_cat[:, _D:]
    eye = jnp.eye(_D, dtype=jnp.float32)

    def expand(mask, dmat):
        n = mask.shape[1]
        return (jnp.asarray(mask)[:, None, :, None] * dmat[None, :, None, :]
                ).reshape(_F * _D, n * _D)

    W0 = (jnp.asarray(_S0)[:, None, :] * w[None, :, None]).reshape(_F * _D, _N0)
    D1L = expand(_SI1, eye * w[:, None])
    D1R = expand(_SJ1, eye)
    W2 = expand(_SI2, A.T) + expand(_SJ2, Bm.T)
    D3L = expand(_SI3, eye * v[:, None])
    D3R = expand(_SJ3, eye)
    SUM1 = jnp.asarray(_S2SEL)
    w_tile = jnp.broadcast_to(w[None, :], (_N2, _D)).reshape(_N2 * _D)
    S2w = SUM1 * w_tile[:, None]

    full = lambda shp: pl.BlockSpec(shp, lambda i: tuple(0 for _ in shp))
    blk = pl.BlockSpec((_BS, _F * _D), lambda i: (i, 0))
    s = pl.pallas_call(
        _interact_kernel,
        grid=(_B // _BS,),
        in_specs=[blk] * 9 + [
            full((_F * _D, _N0)),
            full((_F * _D, _N1 * _D)),
            full((_F * _D, _N1 * _D)),
            full((_F * _D, _N2 * _D)),
            full((_N2 * _D, _N2)),
            full((_F * _D, _N3 * _D)),
            full((_F * _D, _N3 * _D)),
            full((_N1 * _D, _N1)),
        ],
        out_specs=pl.BlockSpec((_BS, _P), lambda i: (i, 0)),
        out_shape=jax.ShapeDtypeStruct((_B, _P), jnp.float32),
    )(*gks, rem416, W0, D1L, D1R, W2, S2w, D3L, D3R, SUM1)

    beta_p = beta[jnp.asarray(_PERM)][None, :]
    out = pl.pallas_call(
        _final_kernel,
        in_specs=[
            pl.BlockSpec((_B, _P), lambda: (0, 0)),
            pl.BlockSpec((1, _P), lambda: (0, 0)),
        ],
        out_specs=pl.BlockSpec((_B, 1), lambda: (0, 0)),
        out_shape=jax.ShapeDtypeStruct((_B, 1), jnp.float32),
    )(s, beta_p)
    return out


# speedup vs baseline: 2.7402x; 1.5499x over previous
import functools

import numpy as np
import jax
import jax.numpy as jnp
from jax.experimental import pallas as pl
from jax.experimental.pallas import tpu as pltpu
from jax.experimental.pallas import tpu_sc as plsc

_B, _F, _V, _D = 4096, 26, 100000, 16
_I, _J = np.triu_indices(_F, k=1)
_P = _I.shape[0]

_T = [np.where(np.arange(_P) % 4 == t)[0] for t in range(4)]
_PERM = np.concatenate(_T)
_N0, _N1, _N2, _N3 = (len(t) for t in _T)

_S0 = np.zeros((_F, _N0), np.float32)
for _q, _p in enumerate(_T[0]):
    _S0[_I[_p], _q] += 1.0
    _S0[_J[_p], _q] += 1.0


def _onehot(plist, sel):
    m = np.zeros((_F, len(plist)), np.float32)
    for q, p in enumerate(plist):
        m[sel[p], q] = 1.0
    return m


_SI1, _SJ1 = _onehot(_T[1], _I), _onehot(_T[1], _J)
_SI2, _SJ2 = _onehot(_T[2], _I), _onehot(_T[2], _J)
_SI3, _SJ3 = _onehot(_T[3], _I), _onehot(_T[3], _J)
_S2SEL = np.zeros((_N2 * _D, _N2), np.float32)
for _q in range(_N2):
    _S2SEL[_q * _D:(_q + 1) * _D, _q] = 1.0

_BS = 256
_NW = 32
_GW = 128
_LANES = _F * 128

_LF = np.arange(_LANES) // 128
_LK = (np.arange(_LANES) % 128) // 16
_LD = np.arange(_LANES) % 16
_REXP = np.zeros((_F, _LANES), np.float32)
_REXP[_LF, np.arange(_LANES)] = 1.0
_KLANE = _LK.astype(np.float32)[None, :]
_CMP = np.zeros((_LANES, _F * 16), np.float32)
_CMP[np.arange(_LANES), _LF * 16 + _LD] = 1.0
_BPW = (_B * _F) // _NW
_NCH = _BPW // _GW
_GD = 128
_RPG = _GD // _D


def _sc_gather(table128, idx3):
    mesh = plsc.VectorSubcoreMesh(core_axis_name="c", subcore_axis_name="s")

    @functools.partial(
        pl.kernel, mesh=mesh,
        out_type=jax.ShapeDtypeStruct((_B * _F, _GD), jnp.float32),
        scratch_types=[
            pltpu.VMEM((_NCH, _GW), jnp.int32),
            pltpu.VMEM((_GW, _GD), jnp.float32),
            pltpu.SemaphoreType.DMA,
        ],
    )
    def kern(table_hbm, idx_hbm, out_hbm, idx_v, rows_v, sem):
        wid = jax.lax.axis_index("s") * 2 + jax.lax.axis_index("c")
        pltpu.sync_copy(idx_hbm.at[wid], idx_v)

        @pl.loop(0, _NCH)
        def _(j):
            pltpu.async_copy(table_hbm.at[idx_v.at[j]], rows_v, sem).wait()
            pltpu.sync_copy(
                rows_v, out_hbm.at[pl.ds(wid * _BPW + j * _GW, _GW)])

    return kern(table128, idx3)


def _interact_kernel(g2_ref, rem_ref, rexp_ref, klane_ref, c_ref,
                     w0_ref, d1l_ref, d1r_ref, w2_ref, s2w_ref,
                     d3l_ref, d3r_ref, sum1_ref, out_ref):
    remf = rem_ref[...].astype(jnp.float32)
    rem_l = jnp.dot(remf, rexp_ref[...])
    mask = (rem_l == klane_ref[...]).astype(jnp.float32)
    eg = g2_ref[...] * mask
    e2 = jnp.dot(eg, c_ref[...])

    s0 = jnp.dot(e2, w0_ref[...])
    a = jnp.dot(e2, d1l_ref[...])
    b = jnp.dot(e2, d1r_ref[...])
    s1 = jnp.dot(a * b, sum1_ref[...])
    c = jnp.maximum(jnp.dot(e2, w2_ref[...]), 0.0)
    s2 = jnp.dot(c, s2w_ref[...])
    p = jnp.dot(e2, d3l_ref[...])
    q = jnp.dot(e2, d3r_ref[...])
    s3 = jnp.dot(p * q, sum1_ref[...])

    out_ref[...] = jnp.concatenate([s0, s1, s2, s3], axis=1)


def _final_kernel(s_ref, beta_ref, out_ref):
    s = s_ref[...]
    mean = jnp.mean(s, axis=0, keepdims=True)
    var = jnp.mean((s - mean) ** 2, axis=0, keepdims=True)
    g = jnp.tanh(beta_ref[...]) / jnp.sqrt(var + 1e-3)
    out_ref[...] = (jnp.sum(s * g, axis=1, keepdims=True)
                    - jnp.sum(g * mean))


def kernel(X, tables, W_cat, W_gen, W_fc, beta):
    table128 = tables.reshape(_F * _V * _D // _GD, _GD)
    phys_idx = (X // _RPG
                + (jnp.arange(_F, dtype=jnp.int32) * (_V // _RPG))[None, :]
                ).reshape(_NW, _NCH, _GW)
    rem = X % _RPG
    g = _sc_gather(table128, phys_idx)
    g2 = g.reshape(_B, _F * _GD)

    w = W_fc[0]
    v = w @ W_gen
    A = W_cat[:, :_D]
    Bm = W_cat[:, _D:]
    eye = jnp.eye(_D, dtype=jnp.float32)

    def expand(mask, dmat):
        n = mask.shape[1]
        return (jnp.asarray(mask)[:, None, :, None] * dmat[None, :, None, :]
                ).reshape(_F * _D, n * _D)

    W0 = (jnp.asarray(_S0)[:, None, :] * w[None, :, None]).reshape(_F * _D, _N0)
    D1L = expand(_SI1, eye * w[:, None])
    D1R = expand(_SJ1, eye)
    W2 = expand(_SI2, A.T) + expand(_SJ2, Bm.T)
    D3L = expand(_SI3, eye * v[:, None])
    D3R = expand(_SJ3, eye)
    SUM1 = jnp.asarray(_S2SEL)
    w_tile = jnp.broadcast_to(w[None, :], (_N2, _D)).reshape(_N2 * _D)
    S2w = SUM1 * w_tile[:, None]

    full = lambda shp: pl.BlockSpec(shp, lambda i: tuple(0 for _ in shp))
    s = pl.pallas_call(
        _interact_kernel,
        grid=(_B // _BS,),
        in_specs=[
            pl.BlockSpec((_BS, _LANES), lambda i: (i, 0)),
            pl.BlockSpec((_BS, _F), lambda i: (i, 0)),
            full((_F, _LANES)),
            full((1, _LANES)),
            full((_LANES, _F * _D)),
            full((_F * _D, _N0)),
            full((_F * _D, _N1 * _D)),
            full((_F * _D, _N1 * _D)),
            full((_F * _D, _N2 * _D)),
            full((_N2 * _D, _N2)),
            full((_F * _D, _N3 * _D)),
            full((_F * _D, _N3 * _D)),
            full((_N1 * _D, _N1)),
        ],
        out_specs=pl.BlockSpec((_BS, _P), lambda i: (i, 0)),
        out_shape=jax.ShapeDtypeStruct((_B, _P), jnp.float32),
    )(g2, rem, jnp.asarray(_REXP), jnp.asarray(_KLANE), jnp.asarray(_CMP),
      W0, D1L, D1R, W2, S2w, D3L, D3R, SUM1)

    beta_p = beta[jnp.asarray(_PERM)][None, :]
    out = pl.pallas_call(
        _final_kernel,
        in_specs=[
            pl.BlockSpec((_B, _P), lambda: (0, 0)),
            pl.BlockSpec((1, _P), lambda: (0, 0)),
        ],
        out_specs=pl.BlockSpec((_B, 1), lambda: (0, 0)),
        out_shape=jax.ShapeDtypeStruct((_B, 1), jnp.float32),
    )(s, beta_p)
    return out


# speedup vs baseline: 2.8554x; 1.0420x over previous
import functools

import numpy as np
import jax
import jax.numpy as jnp
from jax.experimental import pallas as pl
from jax.experimental.pallas import tpu as pltpu
from jax.experimental.pallas import tpu_sc as plsc

_B, _F, _V, _D = 4096, 26, 100000, 16
_I, _J = np.triu_indices(_F, k=1)
_P = _I.shape[0]

_T = [np.where(np.arange(_P) % 4 == t)[0] for t in range(4)]
_PERM = np.concatenate(_T)
_N0, _N1, _N2, _N3 = (len(t) for t in _T)

_S0 = np.zeros((_F, _N0), np.float32)
for _q, _p in enumerate(_T[0]):
    _S0[_I[_p], _q] += 1.0
    _S0[_J[_p], _q] += 1.0


def _onehot(plist, sel):
    m = np.zeros((_F, len(plist)), np.float32)
    for q, p in enumerate(plist):
        m[sel[p], q] = 1.0
    return m


_SI1, _SJ1 = _onehot(_T[1], _I), _onehot(_T[1], _J)
_SI2, _SJ2 = _onehot(_T[2], _I), _onehot(_T[2], _J)
_SI3, _SJ3 = _onehot(_T[3], _I), _onehot(_T[3], _J)
_S2SEL = np.zeros((_N2 * _D, _N2), np.float32)
for _q in range(_N2):
    _S2SEL[_q * _D:(_q + 1) * _D, _q] = 1.0

_BS = 256
_NW = 32
_GW = 128
_LANES = _F * 128

_LF = np.arange(_LANES) // 128
_LK = (np.arange(_LANES) % 128) // 16
_LD = np.arange(_LANES) % 16
_REXP = np.zeros((_F, _LANES), np.float32)
_REXP[_LF, np.arange(_LANES)] = 1.0
_KLANE = _LK.astype(np.float32)[None, :]
_CMP = np.zeros((_LANES, _F * 16), np.float32)
_CMP[np.arange(_LANES), _LF * 16 + _LD] = 1.0
_BPW = (_B * _F) // _NW
_NCH = _BPW // _GW
_GD = 128
_RPG = _GD // _D


def _sc_gather(table128, idx3):
    mesh = plsc.VectorSubcoreMesh(core_axis_name="c", subcore_axis_name="s")

    @functools.partial(
        pl.kernel, mesh=mesh,
        out_type=jax.ShapeDtypeStruct((_B, _F * _GD), jnp.float32),
        scratch_types=[
            pltpu.VMEM((_GW,), jnp.int32),
            pltpu.VMEM((_GW, _GD), jnp.float32),
            pltpu.SemaphoreType.DMA,
        ],
    )
    def kern(table_hbm, idx_hbm, out_hbm, idx_v, rows_v, sem):
        wid = jax.lax.axis_index("s") * 2 + jax.lax.axis_index("c")

        @pl.loop(0, _F)
        def _(f):
            pltpu.sync_copy(idx_hbm.at[f, wid], idx_v)
            pltpu.async_copy(table_hbm.at[idx_v], rows_v, sem).wait()
            pltpu.sync_copy(
                rows_v,
                out_hbm.at[pl.ds(wid * _GW, _GW), pl.ds(f * _GD, _GD)])

    return kern(table128, idx3)


def _interact_kernel(g2_ref, rem_ref, rexp_ref, klane_ref, c_ref,
                     w0_ref, d1l_ref, d1r_ref, w2_ref, s2w_ref,
                     d3l_ref, d3r_ref, sum1_ref, out_ref):
    remf = rem_ref[...].astype(jnp.float32)
    rem_l = jnp.dot(remf, rexp_ref[...])
    mask = (rem_l == klane_ref[...]).astype(jnp.float32)
    eg = g2_ref[...] * mask
    e2 = jnp.dot(eg, c_ref[...])

    s0 = jnp.dot(e2, w0_ref[...])
    a = jnp.dot(e2, d1l_ref[...])
    b = jnp.dot(e2, d1r_ref[...])
    s1 = jnp.dot(a * b, sum1_ref[...])
    c = jnp.maximum(jnp.dot(e2, w2_ref[...]), 0.0)
    s2 = jnp.dot(c, s2w_ref[...])
    p = jnp.dot(e2, d3l_ref[...])
    q = jnp.dot(e2, d3r_ref[...])
    s3 = jnp.dot(p * q, sum1_ref[...])

    out_ref[...] = jnp.concatenate([s0, s1, s2, s3], axis=1)


def _final_kernel(s_ref, beta_ref, out_ref):
    s = s_ref[...]
    mean = jnp.mean(s, axis=0, keepdims=True)
    var = jnp.mean((s - mean) ** 2, axis=0, keepdims=True)
    g = jnp.tanh(beta_ref[...]) / jnp.sqrt(var + 1e-3)
    out_ref[...] = (jnp.sum(s * g, axis=1, keepdims=True)
                    - jnp.sum(g * mean))


def kernel(X, tables, W_cat, W_gen, W_fc, beta):
    table128 = tables.reshape(_F * _V * _D // _GD, _GD)
    phys_idx = (X.T // _RPG
                + (jnp.arange(_F, dtype=jnp.int32) * (_V // _RPG))[:, None]
                ).reshape(_F, _NW, _GW)
    rem = X % _RPG
    g2 = _sc_gather(table128, phys_idx)

    w = W_fc[0]
    v = w @ W_gen
    A = W_cat[:, :_D]
    Bm = W_cat[:, _D:]
    eye = jnp.eye(_D, dtype=jnp.float32)

    def expand(mask, dmat):
        n = mask.shape[1]
        return (jnp.asarray(mask)[:, None, :, None] * dmat[None, :, None, :]
                ).reshape(_F * _D, n * _D)

    W0 = (jnp.asarray(_S0)[:, None, :] * w[None, :, None]).reshape(_F * _D, _N0)
    D1L = expand(_SI1, eye * w[:, None])
    D1R = expand(_SJ1, eye)
    W2 = expand(_SI2, A.T) + expand(_SJ2, Bm.T)
    D3L = expand(_SI3, eye * v[:, None])
    D3R = expand(_SJ3, eye)
    SUM1 = jnp.asarray(_S2SEL)
    w_tile = jnp.broadcast_to(w[None, :], (_N2, _D)).reshape(_N2 * _D)
    S2w = SUM1 * w_tile[:, None]

    full = lambda shp: pl.BlockSpec(shp, lambda i: tuple(0 for _ in shp))
    s = pl.pallas_call(
        _interact_kernel,
        grid=(_B // _BS,),
        in_specs=[
            pl.BlockSpec((_BS, _LANES), lambda i: (i, 0)),
            pl.BlockSpec((_BS, _F), lambda i: (i, 0)),
            full((_F, _LANES)),
            full((1, _LANES)),
            full((_LANES, _F * _D)),
            full((_F * _D, _N0)),
            full((_F * _D, _N1 * _D)),
            full((_F * _D, _N1 * _D)),
            full((_F * _D, _N2 * _D)),
            full((_N2 * _D, _N2)),
            full((_F * _D, _N3 * _D)),
            full((_F * _D, _N3 * _D)),
            full((_N1 * _D, _N1)),
        ],
        out_specs=pl.BlockSpec((_BS, _P), lambda i: (i, 0)),
        out_shape=jax.ShapeDtypeStruct((_B, _P), jnp.float32),
    )(g2, rem, jnp.asarray(_REXP), jnp.asarray(_KLANE), jnp.asarray(_CMP),
      W0, D1L, D1R, W2, S2w, D3L, D3R, SUM1)

    beta_p = beta[jnp.asarray(_PERM)][None, :]
    out = pl.pallas_call(
        _final_kernel,
        in_specs=[
            pl.BlockSpec((_B, _P), lambda: (0, 0)),
            pl.BlockSpec((1, _P), lambda: (0, 0)),
        ],
        out_specs=pl.BlockSpec((_B, 1), lambda: (0, 0)),
        out_shape=jax.ShapeDtypeStruct((_B, 1), jnp.float32),
    )(s, beta_p)
    return out


# speedup vs baseline: 2.9154x; 1.0210x over previous
import functools

import numpy as np
import jax
import jax.numpy as jnp
from jax.experimental import pallas as pl
from jax.experimental.pallas import tpu as pltpu
from jax.experimental.pallas import tpu_sc as plsc

_B, _F, _V, _D = 4096, 26, 100000, 16
_I, _J = np.triu_indices(_F, k=1)
_P = _I.shape[0]

_T = [np.where(np.arange(_P) % 4 == t)[0] for t in range(4)]
_PERM = np.concatenate(_T)
_N0, _N1, _N2, _N3 = (len(t) for t in _T)

_S0 = np.zeros((_F, _N0), np.float32)
for _q, _p in enumerate(_T[0]):
    _S0[_I[_p], _q] += 1.0
    _S0[_J[_p], _q] += 1.0


def _onehot(plist, sel):
    m = np.zeros((_F, len(plist)), np.float32)
    for q, p in enumerate(plist):
        m[sel[p], q] = 1.0
    return m


_SI1, _SJ1 = _onehot(_T[1], _I), _onehot(_T[1], _J)
_SI2, _SJ2 = _onehot(_T[2], _I), _onehot(_T[2], _J)
_SI3, _SJ3 = _onehot(_T[3], _I), _onehot(_T[3], _J)
_S2SEL = np.zeros((_N2 * _D, _N2), np.float32)
for _q in range(_N2):
    _S2SEL[_q * _D:(_q + 1) * _D, _q] = 1.0

_BS = 256
_NW = 32
_GW = 128
_LANES = _F * 128

_LF = np.arange(_LANES) // 128
_LK = (np.arange(_LANES) % 128) // 16
_LD = np.arange(_LANES) % 16
_REXP = np.zeros((_F, _LANES), np.float32)
_REXP[_LF, np.arange(_LANES)] = 1.0
_KLANE = _LK.astype(np.float32)[None, :]
_CMP = np.zeros((_LANES, _F * 16), np.float32)
_CMP[np.arange(_LANES), _LF * 16 + _LD] = 1.0
_BPW = (_B * _F) // _NW
_NCH = _BPW // _GW
_GD = 128
_RPG = _GD // _D


_VG = _V // _RPG
_VGP = 12512


_DC = 8


_VC = 1024
_NVC = _V // _VC
_VREST = _V - _NVC * _VC


def _retile_kernel(tt_ref, out_ref):
    def interleave(xc, nv):
        x3 = jnp.transpose(xc).reshape(nv // _RPG, _RPG, _D)
        return jnp.concatenate([x3[:, s, :] for s in range(_RPG)], axis=1)

    @pl.loop(0, _NVC)
    def _(c):
        xc = tt_ref[0, :, pl.ds(_VC * c, _VC)]
        out_ref[0, pl.ds((_VC // _RPG) * c, _VC // _RPG), :] = (
            interleave(xc, _VC))

    xc = tt_ref[0, :, _NVC * _VC:_V]
    out_ref[0, (_VC // _RPG) * _NVC:_VG, :] = interleave(xc, _VREST)


def _retile(tables):
    tt = jnp.transpose(tables, (0, 2, 1))
    out = pl.pallas_call(
        _retile_kernel,
        grid=(_F,),
        in_specs=[pl.BlockSpec((1, _D, _V), lambda f: (f, 0, 0))],
        out_specs=pl.BlockSpec((1, _VGP, _GD), lambda f: (f, 0, 0)),
        out_shape=jax.ShapeDtypeStruct((_F, _VGP, _GD), jnp.float32),
    )(tt)
    return out.reshape(_F * _VGP, _GD)


def _sc_gather(table128, idx3):
    mesh = plsc.VectorSubcoreMesh(core_axis_name="c", subcore_axis_name="s")

    @functools.partial(
        pl.kernel, mesh=mesh,
        out_type=jax.ShapeDtypeStruct((_B, _F * _GD), jnp.float32),
        scratch_types=[
            pltpu.VMEM((_GW,), jnp.int32),
            pltpu.VMEM((_GW, _GD), jnp.float32),
            pltpu.SemaphoreType.DMA,
        ],
    )
    def kern(table_hbm, idx_hbm, out_hbm, idx_v, rows_v, sem):
        wid = jax.lax.axis_index("s") * 2 + jax.lax.axis_index("c")

        @pl.loop(0, _F)
        def _(f):
            pltpu.sync_copy(idx_hbm.at[f, wid], idx_v)
            pltpu.async_copy(table_hbm.at[idx_v], rows_v, sem).wait()
            pltpu.sync_copy(
                rows_v,
                out_hbm.at[pl.ds(wid * _GW, _GW), pl.ds(f * _GD, _GD)])

    return kern(table128, idx3)


def _interact_kernel(g2_ref, rem_ref, rexp_ref, klane_ref, c_ref,
                     w0_ref, d1l_ref, d1r_ref, w2_ref, s2w_ref,
                     d3l_ref, d3r_ref, sum1_ref, out_ref):
    remf = rem_ref[...].astype(jnp.float32)
    rem_l = jnp.dot(remf, rexp_ref[...])
    mask = (rem_l == klane_ref[...]).astype(jnp.float32)
    eg = g2_ref[...] * mask
    e2 = jnp.dot(eg, c_ref[...])

    s0 = jnp.dot(e2, w0_ref[...])
    a = jnp.dot(e2, d1l_ref[...])
    b = jnp.dot(e2, d1r_ref[...])
    s1 = jnp.dot(a * b, sum1_ref[...])
    c = jnp.maximum(jnp.dot(e2, w2_ref[...]), 0.0)
    s2 = jnp.dot(c, s2w_ref[...])
    p = jnp.dot(e2, d3l_ref[...])
    q = jnp.dot(e2, d3r_ref[...])
    s3 = jnp.dot(p * q, sum1_ref[...])

    out_ref[...] = jnp.concatenate([s0, s1, s2, s3], axis=1)


def _final_kernel(s_ref, beta_ref, out_ref):
    s = s_ref[...]
    mean = jnp.mean(s, axis=0, keepdims=True)
    var = jnp.mean((s - mean) ** 2, axis=0, keepdims=True)
    g = jnp.tanh(beta_ref[...]) / jnp.sqrt(var + 1e-3)
    out_ref[...] = (jnp.sum(s * g, axis=1, keepdims=True)
                    - jnp.sum(g * mean))


def kernel(X, tables, W_cat, W_gen, W_fc, beta):
    table128 = _retile(tables)
    phys_idx = (X.T // _RPG
                + (jnp.arange(_F, dtype=jnp.int32) * _VGP)[:, None]
                ).reshape(_F, _NW, _GW)
    rem = X % _RPG
    g2 = _sc_gather(table128, phys_idx)

    w = W_fc[0]
    v = w @ W_gen
    A = W_cat[:, :_D]
    Bm = W_cat[:, _D:]
    eye = jnp.eye(_D, dtype=jnp.float32)

    def expand(mask, dmat):
        n = mask.shape[1]
        return (jnp.asarray(mask)[:, None, :, None] * dmat[None, :, None, :]
                ).reshape(_F * _D, n * _D)

    W0 = (jnp.asarray(_S0)[:, None, :] * w[None, :, None]).reshape(_F * _D, _N0)
    D1L = expand(_SI1, eye * w[:, None])
    D1R = expand(_SJ1, eye)
    W2 = expand(_SI2, A.T) + expand(_SJ2, Bm.T)
    D3L = expand(_SI3, eye * v[:, None])
    D3R = expand(_SJ3, eye)
    SUM1 = jnp.asarray(_S2SEL)
    w_tile = jnp.broadcast_to(w[None, :], (_N2, _D)).reshape(_N2 * _D)
    S2w = SUM1 * w_tile[:, None]

    full = lambda shp: pl.BlockSpec(shp, lambda i: tuple(0 for _ in shp))
    s = pl.pallas_call(
        _interact_kernel,
        grid=(_B // _BS,),
        in_specs=[
            pl.BlockSpec((_BS, _LANES), lambda i: (i, 0)),
            pl.BlockSpec((_BS, _F), lambda i: (i, 0)),
            full((_F, _LANES)),
            full((1, _LANES)),
            full((_LANES, _F * _D)),
            full((_F * _D, _N0)),
            full((_F * _D, _N1 * _D)),
            full((_F * _D, _N1 * _D)),
            full((_F * _D, _N2 * _D)),
            full((_N2 * _D, _N2)),
            full((_F * _D, _N3 * _D)),
            full((_F * _D, _N3 * _D)),
            full((_N1 * _D, _N1)),
        ],
        out_specs=pl.BlockSpec((_BS, _P), lambda i: (i, 0)),
        out_shape=jax.ShapeDtypeStruct((_B, _P), jnp.float32),
    )(g2, rem, jnp.asarray(_REXP), jnp.asarray(_KLANE), jnp.asarray(_CMP),
      W0, D1L, D1R, W2, S2w, D3L, D3R, SUM1)

    beta_p = beta[jnp.asarray(_PERM)][None, :]
    out = pl.pallas_call(
        _final_kernel,
        in_specs=[
            pl.BlockSpec((_B, _P), lambda: (0, 0)),
            pl.BlockSpec((1, _P), lambda: (0, 0)),
        ],
        out_specs=pl.BlockSpec((_B, 1), lambda: (0, 0)),
        out_shape=jax.ShapeDtypeStruct((_B, 1), jnp.float32),
    )(s, beta_p)
    return out


# speedup vs baseline: 3.5234x; 1.2086x over previous
import functools

import numpy as np
import jax
import jax.numpy as jnp
from jax.experimental import pallas as pl
from jax.experimental.pallas import tpu as pltpu
from jax.experimental.pallas import tpu_sc as plsc

_B, _F, _V, _D = 4096, 26, 100000, 16
_I, _J = np.triu_indices(_F, k=1)
_P = _I.shape[0]

_T = [np.where(np.arange(_P) % 4 == t)[0] for t in range(4)]
_PERM = np.concatenate(_T)
_N0, _N1, _N2, _N3 = (len(t) for t in _T)

_S0 = np.zeros((_F, _N0), np.float32)
for _q, _p in enumerate(_T[0]):
    _S0[_I[_p], _q] += 1.0
    _S0[_J[_p], _q] += 1.0


def _onehot(plist, sel):
    m = np.zeros((_F, len(plist)), np.float32)
    for q, p in enumerate(plist):
        m[sel[p], q] = 1.0
    return m


_SI1, _SJ1 = _onehot(_T[1], _I), _onehot(_T[1], _J)
_SI2, _SJ2 = _onehot(_T[2], _I), _onehot(_T[2], _J)
_SI3, _SJ3 = _onehot(_T[3], _I), _onehot(_T[3], _J)
_S2SEL = np.zeros((_N2 * _D, _N2), np.float32)
for _q in range(_N2):
    _S2SEL[_q * _D:(_q + 1) * _D, _q] = 1.0

_BS = 256
_NW = 32
_GW = 128
_LANES = _F * 128

_LF = np.arange(_LANES) // 128
_LK = (np.arange(_LANES) % 128) // 16
_LD = np.arange(_LANES) % 16
_REXP = np.zeros((_F, _LANES), np.float32)
_REXP[_LF, np.arange(_LANES)] = 1.0
_KLANE = _LK.astype(np.float32)[None, :]
_CMP = np.zeros((_LANES, _F * 16), np.float32)
_CMP[np.arange(_LANES), _LF * 16 + _LD] = 1.0
_BPW = (_B * _F) // _NW
_NCH = _BPW // _GW
_GD = 128
_RPG = _GD // _D


_VG = _V // _RPG
_VGP = 12512


_DC = 8


_VC = 8192
_NVC = _V // _VC
_VREST = _V - _NVC * _VC


def _retile_kernel(tt_ref, out_ref):
    def interleave(xc, nv):
        x3 = jnp.transpose(xc).reshape(nv // _RPG, _RPG, _D)
        return jnp.concatenate([x3[:, s, :] for s in range(_RPG)], axis=1)

    @pl.loop(0, _NVC)
    def _(c):
        xc = tt_ref[0, :, pl.ds(_VC * c, _VC)]
        out_ref[0, pl.ds((_VC // _RPG) * c, _VC // _RPG), :] = (
            interleave(xc, _VC))

    xc = tt_ref[0, :, _NVC * _VC:_V]
    out_ref[0, (_VC // _RPG) * _NVC:_VG, :] = interleave(xc, _VREST)


def _retile(tables):
    tt = jnp.transpose(tables, (0, 2, 1))
    out = pl.pallas_call(
        _retile_kernel,
        grid=(_F,),
        in_specs=[pl.BlockSpec((1, _D, _V), lambda f: (f, 0, 0))],
        out_specs=pl.BlockSpec((1, _VGP, _GD), lambda f: (f, 0, 0)),
        out_shape=jax.ShapeDtypeStruct((_F, _VGP, _GD), jnp.float32),
    )(tt)
    return out.reshape(_F * _VGP, _GD)


def _sc_gather(table128, idx3):
    mesh = plsc.VectorSubcoreMesh(core_axis_name="c", subcore_axis_name="s")

    @functools.partial(
        pl.kernel, mesh=mesh,
        out_type=jax.ShapeDtypeStruct((_B, _F * _GD), jnp.float32),
        scratch_types=[
            pltpu.VMEM((_GW,), jnp.int32),
            pltpu.VMEM((_GW, _GD), jnp.float32),
            pltpu.SemaphoreType.DMA,
        ],
    )
    def kern(table_hbm, idx_hbm, out_hbm, idx_v, rows_v, sem):
        wid = jax.lax.axis_index("s") * 2 + jax.lax.axis_index("c")

        @pl.loop(0, _F)
        def _(f):
            pltpu.sync_copy(idx_hbm.at[f, wid], idx_v)
            pltpu.async_copy(table_hbm.at[idx_v], rows_v, sem).wait()
            pltpu.sync_copy(
                rows_v,
                out_hbm.at[pl.ds(wid * _GW, _GW), pl.ds(f * _GD, _GD)])

    return kern(table128, idx3)


def _interact_kernel(g2_ref, rem_ref, rexp_ref, klane_ref, c_ref,
                     w0_ref, d1l_ref, d1r_ref, w2_ref, s2w_ref,
                     d3l_ref, d3r_ref, sum1_ref, out_ref):
    remf = rem_ref[...].astype(jnp.float32)
    rem_l = jnp.dot(remf, rexp_ref[...])
    mask = (rem_l == klane_ref[...]).astype(jnp.float32)
    eg = g2_ref[...] * mask
    e2 = jnp.dot(eg, c_ref[...])

    s0 = jnp.dot(e2, w0_ref[...])
    a = jnp.dot(e2, d1l_ref[...])
    b = jnp.dot(e2, d1r_ref[...])
    s1 = jnp.dot(a * b, sum1_ref[...])
    c = jnp.maximum(jnp.dot(e2, w2_ref[...]), 0.0)
    s2 = jnp.dot(c, s2w_ref[...])
    p = jnp.dot(e2, d3l_ref[...])
    q = jnp.dot(e2, d3r_ref[...])
    s3 = jnp.dot(p * q, sum1_ref[...])

    out_ref[...] = jnp.concatenate([s0, s1, s2, s3], axis=1)


def _final_kernel(s_ref, beta_ref, out_ref):
    s = s_ref[...]
    mean = jnp.mean(s, axis=0, keepdims=True)
    var = jnp.mean((s - mean) ** 2, axis=0, keepdims=True)
    g = jnp.tanh(beta_ref[...]) / jnp.sqrt(var + 1e-3)
    out_ref[...] = (jnp.sum(s * g, axis=1, keepdims=True)
                    - jnp.sum(g * mean))


def kernel(X, tables, W_cat, W_gen, W_fc, beta):
    table128 = _retile(tables)
    phys_idx = (X.T // _RPG
                + (jnp.arange(_F, dtype=jnp.int32) * _VGP)[:, None]
                ).reshape(_F, _NW, _GW)
    rem = X % _RPG
    g2 = _sc_gather(table128, phys_idx)

    w = W_fc[0]
    v = w @ W_gen
    A = W_cat[:, :_D]
    Bm = W_cat[:, _D:]
    eye = jnp.eye(_D, dtype=jnp.float32)

    def expand(mask, dmat):
        n = mask.shape[1]
        return (jnp.asarray(mask)[:, None, :, None] * dmat[None, :, None, :]
                ).reshape(_F * _D, n * _D)

    W0 = (jnp.asarray(_S0)[:, None, :] * w[None, :, None]).reshape(_F * _D, _N0)
    D1L = expand(_SI1, eye * w[:, None])
    D1R = expand(_SJ1, eye)
    W2 = expand(_SI2, A.T) + expand(_SJ2, Bm.T)
    D3L = expand(_SI3, eye * v[:, None])
    D3R = expand(_SJ3, eye)
    SUM1 = jnp.asarray(_S2SEL)
    w_tile = jnp.broadcast_to(w[None, :], (_N2, _D)).reshape(_N2 * _D)
    S2w = SUM1 * w_tile[:, None]

    full = lambda shp: pl.BlockSpec(shp, lambda i: tuple(0 for _ in shp))
    s = pl.pallas_call(
        _interact_kernel,
        grid=(_B // _BS,),
        in_specs=[
            pl.BlockSpec((_BS, _LANES), lambda i: (i, 0)),
            pl.BlockSpec((_BS, _F), lambda i: (i, 0)),
            full((_F, _LANES)),
            full((1, _LANES)),
            full((_LANES, _F * _D)),
            full((_F * _D, _N0)),
            full((_F * _D, _N1 * _D)),
            full((_F * _D, _N1 * _D)),
            full((_F * _D, _N2 * _D)),
            full((_N2 * _D, _N2)),
            full((_F * _D, _N3 * _D)),
            full((_F * _D, _N3 * _D)),
            full((_N1 * _D, _N1)),
        ],
        out_specs=pl.BlockSpec((_BS, _P), lambda i: (i, 0)),
        out_shape=jax.ShapeDtypeStruct((_B, _P), jnp.float32),
    )(g2, rem, jnp.asarray(_REXP), jnp.asarray(_KLANE), jnp.asarray(_CMP),
      W0, D1L, D1R, W2, S2w, D3L, D3R, SUM1)

    beta_p = beta[jnp.asarray(_PERM)][None, :]
    out = pl.pallas_call(
        _final_kernel,
        in_specs=[
            pl.BlockSpec((_B, _P), lambda: (0, 0)),
            pl.BlockSpec((1, _P), lambda: (0, 0)),
        ],
        out_specs=pl.BlockSpec((_B, 1), lambda: (0, 0)),
        out_shape=jax.ShapeDtypeStruct((_B, 1), jnp.float32),
    )(s, beta_p)
    return out
